# Initial kernel scaffold; baseline (speedup 1.0000x reference)
import functools

import jax
import jax.numpy as jnp
from jax import lax
from jax.experimental import pallas as pl
from jax.experimental.pallas import tpu as pltpu
from jax.experimental.pallas import tpu_sc as plsc

_NC = 2
_NS = 16
_NW = _NC * _NS

_B = 16384
_D = 5
_BPW = _B // _NW
_CHUNK = 128
_NCHUNK = _BPW // _CHUNK


@functools.partial(
    pl.kernel,
    out_type=jax.ShapeDtypeStruct((_B, _D), jnp.float32),
    mesh=plsc.VectorSubcoreMesh(core_axis_name="c", subcore_axis_name="s"),
    scratch_types=[
        pltpu.VMEM((_BPW,), jnp.int32),
        pltpu.VMEM((_BPW, _D), jnp.float32),
        pltpu.SemaphoreType.DMA,
    ],
)
def _embed(idx_hbm, table_hbm, out_hbm, idx_v, rows_v, sem):
    wid = lax.axis_index("s") * _NC + lax.axis_index("c")
    base = wid * _BPW
    pltpu.sync_copy(idx_hbm.at[pl.ds(base, _BPW)], idx_v)
    copies = []
    for c in range(_NCHUNK):
        copies.append(
            pltpu.async_copy(
                table_hbm.at[idx_v.at[pl.ds(c * _CHUNK, _CHUNK)]],
                rows_v.at[pl.ds(c * _CHUNK, _CHUNK)],
                sem,
            )
        )
    for cp in copies:
        cp.wait()
    pltpu.sync_copy(rows_v, out_hbm.at[pl.ds(base, _BPW)])


def kernel(indices, table):
    return _embed(indices.astype(jnp.int32), table)


# speedup vs baseline: 2.0019x; 2.0019x over previous
import functools

import jax
import jax.numpy as jnp
from jax import lax
from jax.experimental import pallas as pl
from jax.experimental.pallas import tpu as pltpu
from jax.experimental.pallas import tpu_sc as plsc

_NC = 2
_NS = 16
_NW = _NC * _NS
_L = 16

_B = 16384
_D = 5
_V = 80
_BPW = _B // _NW
_NG = _BPW // _L


@functools.partial(
    pl.kernel,
    out_type=jax.ShapeDtypeStruct((_B * _D,), jnp.float32),
    mesh=plsc.VectorSubcoreMesh(core_axis_name="c", subcore_axis_name="s"),
    compiler_params=pltpu.CompilerParams(needs_layout_passes=False),
    scratch_types=[
        pltpu.VMEM((_BPW,), jnp.int32),
        pltpu.VMEM((_V * _D,), jnp.float32),
        pltpu.VMEM((_BPW * _D,), jnp.float32),
        pltpu.SemaphoreType.DMA,
        pltpu.SemaphoreType.DMA,
    ],
)
def _embed(idx_hbm, tablet_hbm, out_hbm, idx_v, tablet_v, rows_v, sem_t, sem_i):
    wid = lax.axis_index("s") * _NC + lax.axis_index("c")
    base = wid * _BPW
    ct = pltpu.async_copy(tablet_hbm, tablet_v, sem_t)
    ci = pltpu.async_copy(idx_hbm.at[pl.ds(base, _BPW)], idx_v, sem_i)
    ct.wait()
    ci.wait()
    lane5 = lax.iota(jnp.int32, _L) * _D
    lane5c = [lane5 + c for c in range(_D)]
    cols = [tablet_v.at[pl.ds(c * _V, _V)] for c in range(_D)]
    _Q = 6
    pending = []
    iv = idx_v[pl.ds(0, _L)]
    for g in range(_NG):
        iv_cur = iv
        if g + 1 < _NG:
            iv = idx_v[pl.ds((g + 1) * _L, _L)]
        grp = rows_v.at[pl.ds(g * _L * _D, _L * _D)]
        for c in range(_D):
            vals = plsc.load_gather(cols[c], [iv_cur])
            pending.append((grp, lane5c[c], vals))
            if len(pending) > _Q:
                pgrp, pidx, pvals = pending.pop(0)
                plsc.store_scatter(pgrp, [pidx], pvals)
    for pgrp, pidx, pvals in pending:
        plsc.store_scatter(pgrp, [pidx], pvals)
    pltpu.sync_copy(rows_v, out_hbm.at[pl.ds(base * _D, _BPW * _D)])


def kernel(indices, table):
    out = _embed(indices.astype(jnp.int32), table.T.reshape(-1))
    return out.reshape(_B, _D)


# speedup vs baseline: 2.0743x; 1.0362x over previous
import functools

import jax
import jax.numpy as jnp
from jax import lax
from jax.experimental import pallas as pl
from jax.experimental.pallas import tpu as pltpu
from jax.experimental.pallas import tpu_sc as plsc

_NC = 1
_NS = 16
_NW = _NC * _NS
_L = 16

_B = 16384
_D = 5
_V = 80
_BPW = _B // _NW
_NG = _BPW // _L


@functools.partial(
    pl.kernel,
    out_type=jax.ShapeDtypeStruct((_B * _D,), jnp.float32),
    mesh=plsc.VectorSubcoreMesh(
        core_axis_name="c", subcore_axis_name="s", num_cores=_NC
    ),
    compiler_params=pltpu.CompilerParams(needs_layout_passes=False),
    scratch_types=[
        pltpu.VMEM((_BPW,), jnp.int32),
        pltpu.VMEM((_V * _D,), jnp.float32),
        pltpu.VMEM((_BPW * _D,), jnp.float32),
        pltpu.SemaphoreType.DMA,
        pltpu.SemaphoreType.DMA,
    ],
)
def _embed(idx_hbm, tablet_hbm, out_hbm, idx_v, tablet_v, rows_v, sem_t, sem_i):
    wid = lax.axis_index("s") * _NC + lax.axis_index("c")
    base = wid * _BPW
    ct = pltpu.async_copy(tablet_hbm, tablet_v, sem_t)
    ci = pltpu.async_copy(idx_hbm.at[pl.ds(base, _BPW)], idx_v, sem_i)
    ct.wait()
    ci.wait()
    lane5 = lax.iota(jnp.int32, _L) * _D
    lane5c = [lane5 + c for c in range(_D)]
    cols = [tablet_v.at[pl.ds(c * _V, _V)] for c in range(_D)]
    _Q = 6
    pending = []
    iv = idx_v[pl.ds(0, _L)]
    for g in range(_NG):
        iv_cur = iv
        if g + 1 < _NG:
            iv = idx_v[pl.ds((g + 1) * _L, _L)]
        grp = rows_v.at[pl.ds(g * _L * _D, _L * _D)]
        for c in range(_D):
            vals = plsc.load_gather(cols[c], [iv_cur])
            pending.append((grp, lane5c[c], vals))
            if len(pending) > _Q:
                pgrp, pidx, pvals = pending.pop(0)
                plsc.store_scatter(pgrp, [pidx], pvals)
    for pgrp, pidx, pvals in pending:
        plsc.store_scatter(pgrp, [pidx], pvals)
    pltpu.sync_copy(rows_v, out_hbm.at[pl.ds(base * _D, _BPW * _D)])


def kernel(indices, table):
    out = _embed(indices.astype(jnp.int32), table.T.reshape(-1))
    return out.reshape(_B, _D)


# speedup vs baseline: 2.0765x; 1.0011x over previous
import functools

import jax
import jax.numpy as jnp
from jax import lax
from jax.experimental import pallas as pl
from jax.experimental.pallas import tpu as pltpu
from jax.experimental.pallas import tpu_sc as plsc

_NC = 1
_NS = 16
_NW = _NC * _NS
_L = 16

_B = 16384
_D = 5
_V = 80
_BPW = _B // _NW
_NG = _BPW // _L


@functools.partial(
    pl.kernel,
    out_type=jax.ShapeDtypeStruct((_B * _D,), jnp.float32),
    mesh=plsc.VectorSubcoreMesh(
        core_axis_name="c", subcore_axis_name="s", num_cores=_NC
    ),
    compiler_params=pltpu.CompilerParams(
        needs_layout_passes=False,
        skip_device_barrier=True,
        disable_bounds_checks=True,
        disable_semaphore_checks=True,
    ),
    scratch_types=[
        pltpu.VMEM((_BPW,), jnp.int32),
        pltpu.VMEM((_V * _D,), jnp.float32),
        pltpu.VMEM((_BPW * _D,), jnp.float32),
        pltpu.SemaphoreType.DMA,
        pltpu.SemaphoreType.DMA,
    ],
)
def _embed(idx_hbm, tablet_hbm, out_hbm, idx_v, tablet_v, rows_v, sem_t, sem_i):
    wid = lax.axis_index("s") * _NC + lax.axis_index("c")
    base = wid * _BPW
    ct = pltpu.async_copy(tablet_hbm, tablet_v, sem_t)
    ci = pltpu.async_copy(idx_hbm.at[pl.ds(base, _BPW)], idx_v, sem_i)
    ct.wait()
    ci.wait()
    lane5 = lax.iota(jnp.int32, _L) * _D
    lane5c = [lane5 + c for c in range(_D)]
    cols = [tablet_v.at[pl.ds(c * _V, _V)] for c in range(_D)]
    _Q = 6
    pending = []
    iv = idx_v[pl.ds(0, _L)]
    for g in range(_NG):
        iv_cur = iv
        if g + 1 < _NG:
            iv = idx_v[pl.ds((g + 1) * _L, _L)]
        grp = rows_v.at[pl.ds(g * _L * _D, _L * _D)]
        for c in range(_D):
            vals = plsc.load_gather(cols[c], [iv_cur])
            pending.append((grp, lane5c[c], vals))
            if len(pending) > _Q:
                pgrp, pidx, pvals = pending.pop(0)
                plsc.store_scatter(pgrp, [pidx], pvals)
    for pgrp, pidx, pvals in pending:
        plsc.store_scatter(pgrp, [pidx], pvals)
    pltpu.sync_copy(rows_v, out_hbm.at[pl.ds(base * _D, _BPW * _D)])


def kernel(indices, table):
    out = _embed(indices.astype(jnp.int32), table.T.reshape(-1))
    return out.reshape(_B, _D)


# speedup vs baseline: 2.0837x; 1.0035x over previous
import functools

import jax
import jax.numpy as jnp
from jax import lax
from jax.experimental import pallas as pl
from jax.experimental.pallas import tpu as pltpu
from jax.experimental.pallas import tpu_sc as plsc

_NC = 1
_NS = 16
_NW = _NC * _NS
_L = 16

_B = 16384
_D = 5
_V = 80
_BPW = _B // _NW
_NG = _BPW // _L


@functools.partial(
    pl.kernel,
    out_type=jax.ShapeDtypeStruct((_B * _D,), jnp.float32),
    mesh=plsc.VectorSubcoreMesh(
        core_axis_name="c", subcore_axis_name="s", num_cores=_NC
    ),
    compiler_params=pltpu.CompilerParams(
        needs_layout_passes=False,
        skip_device_barrier=True,
        disable_bounds_checks=True,
        disable_semaphore_checks=True,
    ),
    scratch_types=[
        pltpu.VMEM((_BPW,), jnp.int32),
        pltpu.VMEM((_V * _D,), jnp.float32),
        pltpu.VMEM((_BPW * _D,), jnp.float32),
        pltpu.SemaphoreType.DMA,
        pltpu.SemaphoreType.DMA,
    ],
)
def _embed(idx_hbm, tablet_hbm, out_hbm, idx_v, tablet_v, rows_v, sem_t, sem_i):
    wid = lax.axis_index("s") * _NC + lax.axis_index("c")
    base = wid * _BPW
    ct = pltpu.async_copy(tablet_hbm, tablet_v, sem_t)
    ci = pltpu.async_copy(idx_hbm.at[pl.ds(base, _BPW)], idx_v, sem_i)
    ct.wait()
    ci.wait()
    lane5 = lax.iota(jnp.int32, _L) * _D
    lane5c = [lane5 + c for c in range(_D)]
    cols = [tablet_v.at[pl.ds(c * _V, _V)] for c in range(_D)]
    _Q = 6
    _HG = _NG // 2
    _HW = _HG * _L * _D
    pending = []
    out_copies = []
    iv = idx_v[pl.ds(0, _L)]
    for g in range(_NG):
        iv_cur = iv
        if g + 1 < _NG:
            iv = idx_v[pl.ds((g + 1) * _L, _L)]
        grp = rows_v.at[pl.ds(g * _L * _D, _L * _D)]
        for c in range(_D):
            vals = plsc.load_gather(cols[c], [iv_cur])
            pending.append((grp, lane5c[c], vals))
            if len(pending) > _Q:
                pgrp, pidx, pvals = pending.pop(0)
                plsc.store_scatter(pgrp, [pidx], pvals)
        if g == _HG + 1:
            out_copies.append(
                pltpu.async_copy(
                    rows_v.at[pl.ds(0, _HW)],
                    out_hbm.at[pl.ds(base * _D, _HW)],
                    sem_t,
                )
            )
    for pgrp, pidx, pvals in pending:
        plsc.store_scatter(pgrp, [pidx], pvals)
    out_copies.append(
        pltpu.async_copy(
            rows_v.at[pl.ds(_HW, _BPW * _D - _HW)],
            out_hbm.at[pl.ds(base * _D + _HW, _BPW * _D - _HW)],
            sem_i,
        )
    )
    for cp in out_copies:
        cp.wait()


def kernel(indices, table):
    out = _embed(indices.astype(jnp.int32), table.T.reshape(-1))
    return out.reshape(_B, _D)


# speedup vs baseline: 2.2737x; 1.0912x over previous
import functools

import jax
import jax.numpy as jnp
from jax import lax
from jax.experimental import pallas as pl
from jax.experimental.pallas import tpu as pltpu
from jax.experimental.pallas import tpu_sc as plsc

_NC = 1
_NS = 16
_NW = _NC * _NS
_L = 16

_B = 16384
_D = 5
_V = 80
_BPW = _B // _NW
_NG = _BPW // _L


@functools.partial(
    pl.kernel,
    out_type=jax.ShapeDtypeStruct((_B, _D), jnp.float32),
    mesh=plsc.VectorSubcoreMesh(
        core_axis_name="c", subcore_axis_name="s", num_cores=_NC
    ),
    compiler_params=pltpu.CompilerParams(
        needs_layout_passes=False,
        use_tc_tiling_on_sc=False,
        skip_device_barrier=True,
        disable_bounds_checks=True,
        disable_semaphore_checks=True,
    ),
    scratch_types=[
        pltpu.VMEM((_BPW,), jnp.int32),
        pltpu.VMEM((_V * _D,), jnp.float32),
        pltpu.VMEM((_BPW, _D), jnp.float32),
        pltpu.SemaphoreType.DMA,
        pltpu.SemaphoreType.DMA,
    ],
)
def _embed(idx_hbm, tablet_hbm, out_hbm, idx_v, tablet_v, rows_v, sem_t, sem_i):
    wid = lax.axis_index("s") * _NC + lax.axis_index("c")
    base = wid * _BPW
    ct = pltpu.async_copy(tablet_hbm, tablet_v, sem_t)
    ci = pltpu.async_copy(idx_hbm.at[pl.ds(base, _BPW)], idx_v, sem_i)
    ct.wait()
    ci.wait()
    lane = lax.iota(jnp.int32, _L)
    colc = [jnp.full((_L,), c, jnp.int32) for c in range(_D)]
    cols = [tablet_v.at[pl.ds(c * _V, _V)] for c in range(_D)]
    _Q = 6
    _HG = _NG // 2
    _HR = _HG * _L
    pending = []
    out_copies = []
    iv = idx_v[pl.ds(0, _L)]
    for g in range(_NG):
        iv_cur = iv
        if g + 1 < _NG:
            iv = idx_v[pl.ds((g + 1) * _L, _L)]
        grp = rows_v.at[pl.ds(g * _L, _L), :]
        for c in range(_D):
            vals = plsc.load_gather(cols[c], [iv_cur])
            pending.append((grp, colc[c], vals))
            if len(pending) > _Q:
                pgrp, pcol, pvals = pending.pop(0)
                plsc.store_scatter(pgrp, [lane, pcol], pvals)
        if g == _HG + 1:
            out_copies.append(
                pltpu.async_copy(
                    rows_v.at[pl.ds(0, _HR), :],
                    out_hbm.at[pl.ds(base, _HR), :],
                    sem_t,
                )
            )
    for pgrp, pcol, pvals in pending:
        plsc.store_scatter(pgrp, [lane, pcol], pvals)
    out_copies.append(
        pltpu.async_copy(
            rows_v.at[pl.ds(_HR, _BPW - _HR), :],
            out_hbm.at[pl.ds(base + _HR, _BPW - _HR), :],
            sem_i,
        )
    )
    for cp in out_copies:
        cp.wait()


def kernel(indices, table):
    return _embed(indices.astype(jnp.int32), table.T.reshape(-1))
